# baseline (speedup 1.0000x reference)
import functools

import jax
import jax.numpy as jnp
from jax import lax
from jax.experimental import pallas as pl
from jax.experimental.pallas import tpu as pltpu
from jax.experimental.pallas import tpu_sc as plsc

VOCAB = 100000
D_HALF = 64
D_MODEL = 128
B_DIM = 4096
L_DIM = 50
N_TOKENS = B_DIM * L_DIM
BQ = B_DIM // 4
OCT_TOK = 8 * B_DIM

NW = 32
IDXW = 128

CHUNK_QUADS = (1, 2, 4, 4, 2)
CHUNK_L0 = (0, 4, 12, 28, 44)
CHUNK_TOKS = (16384, 32768, 65536, 65536, 24576)


def _table_transform_tc(emb_table_T, W_comb):
    BR = 4096
    grid = ((VOCAB + BR - 1) // BR,)

    def body(t_ref, w_ref, o_ref):
        o_ref[...] = jax.lax.dot_general(
            t_ref[...], w_ref[:D_HALF, :],
            dimension_numbers=(((0,), (0,)), ((), ())),
            preferred_element_type=jnp.float32)

    return pl.pallas_call(
        body,
        grid=grid,
        in_specs=[
            pl.BlockSpec((D_HALF, BR), lambda i: (0, i)),
            pl.BlockSpec((D_MODEL, D_MODEL), lambda i: (0, 0)),
        ],
        out_specs=pl.BlockSpec((BR, D_MODEL), lambda i: (i, 0)),
        out_shape=jax.ShapeDtypeStruct((VOCAB, D_MODEL), jnp.float32),
    )(emb_table_T, W_comb)


def _gather_sc(table_t, ids_chunk, n_tok):
    bpw = n_tok // NW
    nchunk = bpw // 256
    mesh = plsc.VectorSubcoreMesh(core_axis_name="c", subcore_axis_name="s")

    @functools.partial(
        pl.kernel,
        mesh=mesh,
        out_type=jax.ShapeDtypeStruct((n_tok, D_MODEL), jnp.float32),
        scratch_types=[
            pltpu.VMEM((bpw,), jnp.int32),
            pltpu.VMEM((256, D_MODEL), jnp.float32),
            pltpu.VMEM((256, D_MODEL), jnp.float32),
            pltpu.SemaphoreType.DMA,
            pltpu.SemaphoreType.DMA,
            pltpu.SemaphoreType.DMA,
        ],
    )
    def k(table_hbm, ids_hbm, out_hbm, idx_v, buf0, buf1, gsem, osem0, osem1):
        wid = lax.axis_index("s") * 2 + lax.axis_index("c")
        base = wid * bpw
        pltpu.sync_copy(ids_hbm.at[pl.ds(base, bpw)], idx_v)
        bufs = (buf0, buf1)
        osems = (osem0, osem1)

        def issue(c, buf):
            hs = []
            for j in range(2):
                hs.append(pltpu.async_copy(
                    table_hbm.at[idx_v.at[pl.ds(c * 256 + j * IDXW, IDXW)]],
                    buf.at[pl.ds(j * IDXW, IDXW)],
                    gsem))
            return hs

        pending = issue(0, bufs[0])
        out_h = [None, None]
        for c in range(nchunk):
            b = c & 1
            for h in pending:
                h.wait()
            if c + 1 < nchunk:
                if out_h[1 - b] is not None:
                    out_h[1 - b].wait()
                pending = issue(c + 1, bufs[1 - b])
            out_h[b] = pltpu.async_copy(
                bufs[b], out_hbm.at[pl.ds(base + c * 256, 256)], osems[b])
        out_h[(nchunk - 1) & 1].wait()

    return k(table_t, ids_chunk)


def _dense_tc(gathered_c, stats3, W_stat, b_stat, W_bot, b_comb,
              quad0, nquads, nl_real, prev_out=None):
    grid = (nquads, 4)
    g4 = gathered_c.reshape(nl_real, 4, BQ, D_MODEL)

    def body(*refs):
        if prev_out is None:
            g_ref, st_ref, ws_ref, bs_ref, wc_ref, bc_ref, out_ref = refs
        else:
            _, g_ref, st_ref, ws_ref, bs_ref, wc_ref, bc_ref, out_ref = refs
        li = pl.program_id(0)
        for ll in range(4):
            lidx = jnp.minimum((quad0 + li) * 4 + ll, L_DIM - 1)
            x = st_ref[:, lidx, :]
            s = jax.lax.dot_general(
                x, ws_ref[...],
                dimension_numbers=(((0,), (0,)), ((), ())),
                preferred_element_type=jnp.float32) + bs_ref[...]
            s = 0.5 * s * (1.0 + lax.erf(s * 0.7071067811865476))
            bot = jnp.dot(s, wc_ref[...], preferred_element_type=jnp.float32)
            out_ref[ll, 0] = g_ref[ll, 0] + bot + bc_ref[...]

    in_specs = [
        pl.BlockSpec((4, 1, BQ, D_MODEL), lambda li, bq: (li, bq, 0, 0)),
        pl.BlockSpec((10, L_DIM, BQ), lambda li, bq: (0, 0, bq)),
        pl.BlockSpec((10, D_HALF), lambda li, bq: (0, 0)),
        pl.BlockSpec((1, D_HALF), lambda li, bq: (0, 0)),
        pl.BlockSpec((D_HALF, D_MODEL), lambda li, bq: (0, 0)),
        pl.BlockSpec((1, D_MODEL), lambda li, bq: (0, 0)),
    ]
    out_spec = pl.BlockSpec((4, 1, BQ, D_MODEL),
                            lambda li, bq: (quad0 + li, bq, 0, 0))
    operands = [g4, stats3, W_stat, b_stat.reshape(1, D_HALF),
                W_bot, b_comb.reshape(1, D_MODEL)]
    kwargs = {}
    if prev_out is not None:
        in_specs = [pl.BlockSpec(memory_space=pl.ANY)] + in_specs
        operands = [prev_out] + operands
        kwargs["input_output_aliases"] = {0: 0}

    return pl.pallas_call(
        body,
        grid=grid,
        in_specs=in_specs,
        out_specs=out_spec,
        out_shape=jax.ShapeDtypeStruct((L_DIM, 4, BQ, D_MODEL), jnp.float32),
        **kwargs,
    )(*operands)


def kernel(card_ids, card_stats, emb_table, W_stat, b_stat, W_comb, b_comb):
    B, L = card_ids.shape
    ids_flat = card_ids.T.reshape(N_TOKENS).astype(jnp.int32)
    stats3 = card_stats.transpose(2, 1, 0)
    W_bot = W_comb[D_HALF:, :]

    table_t = _table_transform_tc(emb_table.T, W_comb)

    nchunks = len(CHUNK_TOKS)
    gs = []
    off = 0
    for k in range(nchunks):
        gs.append(_gather_sc(table_t, ids_flat[off:off + CHUNK_TOKS[k]],
                             CHUNK_TOKS[k]))
        off += CHUNK_TOKS[k]

    out = None
    for k in range(nchunks):
        nl_real = CHUNK_TOKS[k] // B_DIM
        out = _dense_tc(gs[k], stats3, W_stat, b_stat, W_bot, b_comb,
                        CHUNK_L0[k] // 4, CHUNK_QUADS[k], nl_real,
                        prev_out=out)

    return out.reshape(L_DIM, B_DIM, D_MODEL).transpose(1, 0, 2)


# speedup vs baseline: 1.1610x; 1.1610x over previous
import functools

import jax
import jax.numpy as jnp
from jax import lax
from jax.experimental import pallas as pl
from jax.experimental.pallas import tpu as pltpu
from jax.experimental.pallas import tpu_sc as plsc

VOCAB = 100000
D_HALF = 64
D_MODEL = 128
B_DIM = 4096
L_DIM = 50
N_TOKENS = B_DIM * L_DIM
BQ = B_DIM // 4
OCT_TOK = 8 * B_DIM

NW = 32
IDXW = 128

CHUNK_OCTS = (1, 2, 2, 2)
CHUNK_L0 = (0, 8, 24, 40)
CHUNK_TOKS = (32768, 65536, 65536, 40960)


def _table_transform_tc(emb_table_T, W_comb):
    BR = 4096
    grid = ((VOCAB + BR - 1) // BR,)

    def body(t_ref, w_ref, o_ref):
        o_ref[...] = jax.lax.dot_general(
            t_ref[...], w_ref[:D_HALF, :],
            dimension_numbers=(((0,), (0,)), ((), ())),
            preferred_element_type=jnp.float32)

    return pl.pallas_call(
        body,
        grid=grid,
        in_specs=[
            pl.BlockSpec((D_HALF, BR), lambda i: (0, i)),
            pl.BlockSpec((D_MODEL, D_MODEL), lambda i: (0, 0)),
        ],
        out_specs=pl.BlockSpec((BR, D_MODEL), lambda i: (i, 0)),
        out_shape=jax.ShapeDtypeStruct((VOCAB, D_MODEL), jnp.float32),
    )(emb_table_T, W_comb)


def _gather_sc(table_t, ids_chunk, n_tok):
    bpw = n_tok // NW
    nchunk = bpw // 256
    mesh = plsc.VectorSubcoreMesh(core_axis_name="c", subcore_axis_name="s")

    @functools.partial(
        pl.kernel,
        mesh=mesh,
        out_type=jax.ShapeDtypeStruct((n_tok, D_MODEL), jnp.float32),
        scratch_types=[
            pltpu.VMEM((bpw,), jnp.int32),
            pltpu.VMEM((256, D_MODEL), jnp.float32),
            pltpu.VMEM((256, D_MODEL), jnp.float32),
            pltpu.VMEM((256, D_MODEL), jnp.float32),
            pltpu.SemaphoreType.DMA,
            pltpu.SemaphoreType.DMA,
            pltpu.SemaphoreType.DMA,
            pltpu.SemaphoreType.DMA,
        ],
    )
    def k(table_hbm, ids_hbm, out_hbm, idx_v, buf0, buf1, buf2,
          gsem, osem0, osem1, osem2):
        wid = lax.axis_index("s") * 2 + lax.axis_index("c")
        base = wid * bpw
        pltpu.sync_copy(ids_hbm.at[pl.ds(base, bpw)], idx_v)
        bufs = (buf0, buf1, buf2)
        osems = (osem0, osem1, osem2)

        def issue(c, buf):
            hs = []
            for j in range(2):
                hs.append(pltpu.async_copy(
                    table_hbm.at[idx_v.at[pl.ds(c * 256 + j * IDXW, IDXW)]],
                    buf.at[pl.ds(j * IDXW, IDXW)],
                    gsem))
            return hs

        pending = [issue(0, bufs[0])]
        if nchunk > 1:
            pending.append(issue(1, bufs[1]))
        out_h = [None, None, None]
        for c in range(nchunk):
            b = c % 3
            for h in pending.pop(0):
                h.wait()
            if c + 2 < nchunk:
                nb = (c + 2) % 3
                if out_h[nb] is not None:
                    out_h[nb].wait()
                    out_h[nb] = None
                pending.append(issue(c + 2, bufs[nb]))
            out_h[b] = pltpu.async_copy(
                bufs[b], out_hbm.at[pl.ds(base + c * 256, 256)], osems[b])
        for h in out_h:
            if h is not None:
                h.wait()

    return k(table_t, ids_chunk)


def _dense_tc(gathered_c, stats3, W_stat, b_stat, W_bot, b_comb,
              oct0, nocts, nl_real, prev_out=None):
    grid = (nocts, 4)
    g4 = gathered_c.reshape(nl_real, 4, BQ, D_MODEL)

    def body(*refs):
        if prev_out is None:
            g_ref, st_ref, ws_ref, bs_ref, wc_ref, bc_ref, out_ref = refs
        else:
            _, g_ref, st_ref, ws_ref, bs_ref, wc_ref, bc_ref, out_ref = refs
        for ll in range(8):
            x = st_ref[:, ll, :]
            s = jax.lax.dot_general(
                x, ws_ref[...],
                dimension_numbers=(((0,), (0,)), ((), ())),
                preferred_element_type=jnp.float32) + bs_ref[...]
            s = 0.5 * s * (1.0 + lax.erf(s * 0.7071067811865476))
            bot = jnp.dot(s, wc_ref[...], preferred_element_type=jnp.float32)
            out_ref[ll, 0] = g_ref[ll, 0] + bot + bc_ref[...]

    in_specs = [
        pl.BlockSpec((8, 1, BQ, D_MODEL), lambda li, bq: (li, bq, 0, 0)),
        pl.BlockSpec((10, 8, BQ), lambda li, bq: (0, oct0 + li, bq)),
        pl.BlockSpec((10, D_HALF), lambda li, bq: (0, 0)),
        pl.BlockSpec((1, D_HALF), lambda li, bq: (0, 0)),
        pl.BlockSpec((D_HALF, D_MODEL), lambda li, bq: (0, 0)),
        pl.BlockSpec((1, D_MODEL), lambda li, bq: (0, 0)),
    ]
    out_spec = pl.BlockSpec((8, 1, BQ, D_MODEL),
                            lambda li, bq: (oct0 + li, bq, 0, 0))
    operands = [g4, stats3, W_stat, b_stat.reshape(1, D_HALF),
                W_bot, b_comb.reshape(1, D_MODEL)]
    kwargs = {}
    if prev_out is not None:
        in_specs = [pl.BlockSpec(memory_space=pl.ANY)] + in_specs
        operands = [prev_out] + operands
        kwargs["input_output_aliases"] = {0: 0}

    return pl.pallas_call(
        body,
        grid=grid,
        in_specs=in_specs,
        out_specs=out_spec,
        out_shape=jax.ShapeDtypeStruct((L_DIM, 4, BQ, D_MODEL), jnp.float32),
        **kwargs,
    )(*operands)


def kernel(card_ids, card_stats, emb_table, W_stat, b_stat, W_comb, b_comb):
    B, L = card_ids.shape
    ids_flat = card_ids.T.reshape(N_TOKENS).astype(jnp.int32)
    stats3 = card_stats.transpose(2, 1, 0)
    W_bot = W_comb[D_HALF:, :]

    table_t = _table_transform_tc(emb_table.T, W_comb)

    nchunks = len(CHUNK_TOKS)
    gs = []
    off = 0
    for k in range(nchunks):
        gs.append(_gather_sc(table_t, ids_flat[off:off + CHUNK_TOKS[k]],
                             CHUNK_TOKS[k]))
        off += CHUNK_TOKS[k]

    out = None
    for k in range(nchunks):
        nl_real = CHUNK_TOKS[k] // B_DIM
        out = _dense_tc(gs[k], stats3, W_stat, b_stat, W_bot, b_comb,
                        CHUNK_L0[k] // 8, CHUNK_OCTS[k], nl_real,
                        prev_out=out)

    return out.reshape(L_DIM, B_DIM, D_MODEL).transpose(1, 0, 2)


# speedup vs baseline: 1.2088x; 1.0412x over previous
import functools

import jax
import jax.numpy as jnp
from jax import lax
from jax.experimental import pallas as pl
from jax.experimental.pallas import tpu as pltpu
from jax.experimental.pallas import tpu_sc as plsc

VOCAB = 100000
D_HALF = 64
D_MODEL = 128
B_DIM = 4096
L_DIM = 50
N_TOKENS = B_DIM * L_DIM
BQ = B_DIM // 4
OCT_TOK = 8 * B_DIM

NW = 32
IDXW = 128

CHUNK_OCTS = (1, 2, 2, 2)
CHUNK_L0 = (0, 8, 24, 40)
CHUNK_TOKS = (32768, 65536, 65536, 40960)


def _table_transform_tc(emb_table_T, W_comb):
    BR = 8192
    grid = ((VOCAB + BR - 1) // BR,)

    def body(t_ref, w_ref, o_ref):
        o_ref[...] = jax.lax.dot_general(
            t_ref[...], w_ref[:D_HALF, :],
            dimension_numbers=(((0,), (0,)), ((), ())),
            preferred_element_type=jnp.float32)

    return pl.pallas_call(
        body,
        grid=grid,
        in_specs=[
            pl.BlockSpec((D_HALF, BR), lambda i: (0, i)),
            pl.BlockSpec((D_MODEL, D_MODEL), lambda i: (0, 0)),
        ],
        out_specs=pl.BlockSpec((BR, D_MODEL), lambda i: (i, 0)),
        out_shape=jax.ShapeDtypeStruct((VOCAB, D_MODEL), jnp.float32),
    )(emb_table_T, W_comb)


def _gather_sc(table_t, ids_chunk, n_tok):
    bpw = n_tok // NW
    nchunk = bpw // 256
    mesh = plsc.VectorSubcoreMesh(core_axis_name="c", subcore_axis_name="s")

    @functools.partial(
        pl.kernel,
        mesh=mesh,
        out_type=jax.ShapeDtypeStruct((n_tok, D_MODEL), jnp.float32),
        scratch_types=[
            pltpu.VMEM((bpw,), jnp.int32),
            pltpu.VMEM((256, D_MODEL), jnp.float32),
            pltpu.VMEM((256, D_MODEL), jnp.float32),
            pltpu.VMEM((256, D_MODEL), jnp.float32),
            pltpu.SemaphoreType.DMA,
            pltpu.SemaphoreType.DMA,
            pltpu.SemaphoreType.DMA,
            pltpu.SemaphoreType.DMA,
        ],
    )
    def k(table_hbm, ids_hbm, out_hbm, idx_v, buf0, buf1, buf2,
          gsem, osem0, osem1, osem2):
        wid = lax.axis_index("s") * 2 + lax.axis_index("c")
        base = wid * bpw
        pltpu.sync_copy(ids_hbm.at[pl.ds(base, bpw)], idx_v)
        bufs = (buf0, buf1, buf2)
        osems = (osem0, osem1, osem2)

        def issue(c, buf):
            hs = []
            for j in range(2):
                hs.append(pltpu.async_copy(
                    table_hbm.at[idx_v.at[pl.ds(c * 256 + j * IDXW, IDXW)]],
                    buf.at[pl.ds(j * IDXW, IDXW)],
                    gsem))
            return hs

        pending = [issue(0, bufs[0])]
        if nchunk > 1:
            pending.append(issue(1, bufs[1]))
        out_h = [None, None, None]
        for c in range(nchunk):
            b = c % 3
            for h in pending.pop(0):
                h.wait()
            if c + 2 < nchunk:
                nb = (c + 2) % 3
                if out_h[nb] is not None:
                    out_h[nb].wait()
                    out_h[nb] = None
                pending.append(issue(c + 2, bufs[nb]))
            out_h[b] = pltpu.async_copy(
                bufs[b], out_hbm.at[pl.ds(base + c * 256, 256)], osems[b])
        for h in out_h:
            if h is not None:
                h.wait()

    return k(table_t, ids_chunk)


def _dense_tc(gathered_c, stats3, W_stat, b_stat, W_bot, b_comb,
              oct0, nocts, nl_real, prev_out=None):
    grid = (nocts, 4)
    g4 = gathered_c.reshape(nl_real, 4, BQ, D_MODEL)

    def body(*refs):
        if prev_out is None:
            g_ref, st_ref, ws_ref, bs_ref, wc_ref, bc_ref, out_ref = refs
        else:
            _, g_ref, st_ref, ws_ref, bs_ref, wc_ref, bc_ref, out_ref = refs
        for ll in range(8):
            x = st_ref[:, ll, :]
            s = jax.lax.dot_general(
                x, ws_ref[...],
                dimension_numbers=(((0,), (0,)), ((), ())),
                preferred_element_type=jnp.float32) + bs_ref[...]
            s = 0.5 * s * (1.0 + lax.erf(s * 0.7071067811865476))
            bot = jnp.dot(s, wc_ref[...], preferred_element_type=jnp.float32)
            out_ref[ll, 0] = g_ref[ll, 0] + bot + bc_ref[...]

    in_specs = [
        pl.BlockSpec((8, 1, BQ, D_MODEL), lambda li, bq: (li, bq, 0, 0)),
        pl.BlockSpec((10, 8, BQ), lambda li, bq: (0, oct0 + li, bq)),
        pl.BlockSpec((10, D_HALF), lambda li, bq: (0, 0)),
        pl.BlockSpec((1, D_HALF), lambda li, bq: (0, 0)),
        pl.BlockSpec((D_HALF, D_MODEL), lambda li, bq: (0, 0)),
        pl.BlockSpec((1, D_MODEL), lambda li, bq: (0, 0)),
    ]
    out_spec = pl.BlockSpec((8, 1, BQ, D_MODEL),
                            lambda li, bq: (oct0 + li, bq, 0, 0))
    operands = [g4, stats3, W_stat, b_stat.reshape(1, D_HALF),
                W_bot, b_comb.reshape(1, D_MODEL)]
    kwargs = {}
    if prev_out is not None:
        in_specs = [pl.BlockSpec(memory_space=pl.ANY)] + in_specs
        operands = [prev_out] + operands
        kwargs["input_output_aliases"] = {0: 0}

    return pl.pallas_call(
        body,
        grid=grid,
        in_specs=in_specs,
        out_specs=out_spec,
        out_shape=jax.ShapeDtypeStruct((L_DIM, 4, BQ, D_MODEL), jnp.float32),
        **kwargs,
    )(*operands)


def kernel(card_ids, card_stats, emb_table, W_stat, b_stat, W_comb, b_comb):
    B, L = card_ids.shape
    ids_flat = card_ids.T.reshape(N_TOKENS).astype(jnp.int32)
    stats3 = card_stats.transpose(2, 1, 0)
    W_bot = W_comb[D_HALF:, :]

    table_t = _table_transform_tc(emb_table.T, W_comb)

    nchunks = len(CHUNK_TOKS)
    gs = []
    off = 0
    for k in range(nchunks):
        gs.append(_gather_sc(table_t, ids_flat[off:off + CHUNK_TOKS[k]],
                             CHUNK_TOKS[k]))
        off += CHUNK_TOKS[k]

    out = None
    for k in range(nchunks):
        nl_real = CHUNK_TOKS[k] // B_DIM
        out = _dense_tc(gs[k], stats3, W_stat, b_stat, W_bot, b_comb,
                        CHUNK_L0[k] // 8, CHUNK_OCTS[k], nl_real,
                        prev_out=out)

    return out.reshape(L_DIM, B_DIM, D_MODEL).transpose(1, 0, 2)


# speedup vs baseline: 1.2198x; 1.0091x over previous
import functools

import jax
import jax.numpy as jnp
from jax import lax
from jax.experimental import pallas as pl
from jax.experimental.pallas import tpu as pltpu
from jax.experimental.pallas import tpu_sc as plsc

VOCAB = 100000
D_HALF = 64
D_MODEL = 128
B_DIM = 4096
L_DIM = 50
N_TOKENS = B_DIM * L_DIM
BQ = B_DIM // 4
OCT_TOK = 8 * B_DIM

NW = 32
IDXW = 128

CHUNK_OCTS = (1, 2, 2, 2)
CHUNK_L0 = (0, 8, 24, 40)
CHUNK_TOKS = (32768, 65536, 65536, 40960)


def _table_transform_tc(emb_table_T, W_comb):
    BR = 12800
    grid = ((VOCAB + BR - 1) // BR,)

    def body(t_ref, w_ref, o_ref):
        o_ref[...] = jax.lax.dot_general(
            t_ref[...], w_ref[:D_HALF, :],
            dimension_numbers=(((0,), (0,)), ((), ())),
            preferred_element_type=jnp.float32)

    return pl.pallas_call(
        body,
        grid=grid,
        in_specs=[
            pl.BlockSpec((D_HALF, BR), lambda i: (0, i)),
            pl.BlockSpec((D_MODEL, D_MODEL), lambda i: (0, 0)),
        ],
        out_specs=pl.BlockSpec((BR, D_MODEL), lambda i: (i, 0)),
        out_shape=jax.ShapeDtypeStruct((VOCAB, D_MODEL), jnp.float32),
    )(emb_table_T, W_comb)


def _gather_sc(table_t, ids_chunk, n_tok):
    bpw = n_tok // NW
    nchunk = bpw // 256
    mesh = plsc.VectorSubcoreMesh(core_axis_name="c", subcore_axis_name="s")

    @functools.partial(
        pl.kernel,
        mesh=mesh,
        out_type=jax.ShapeDtypeStruct((n_tok, D_MODEL), jnp.float32),
        scratch_types=[
            pltpu.VMEM((bpw,), jnp.int32),
            pltpu.VMEM((256, D_MODEL), jnp.float32),
            pltpu.VMEM((256, D_MODEL), jnp.float32),
            pltpu.VMEM((256, D_MODEL), jnp.float32),
            pltpu.SemaphoreType.DMA,
            pltpu.SemaphoreType.DMA,
            pltpu.SemaphoreType.DMA,
            pltpu.SemaphoreType.DMA,
        ],
    )
    def k(table_hbm, ids_hbm, out_hbm, idx_v, buf0, buf1, buf2,
          gsem, osem0, osem1, osem2):
        wid = lax.axis_index("s") * 2 + lax.axis_index("c")
        base = wid * bpw
        pltpu.sync_copy(ids_hbm.at[pl.ds(base, bpw)], idx_v)
        bufs = (buf0, buf1, buf2)
        osems = (osem0, osem1, osem2)

        def issue(c, buf):
            hs = []
            for j in range(2):
                hs.append(pltpu.async_copy(
                    table_hbm.at[idx_v.at[pl.ds(c * 256 + j * IDXW, IDXW)]],
                    buf.at[pl.ds(j * IDXW, IDXW)],
                    gsem))
            return hs

        pending = [issue(0, bufs[0])]
        if nchunk > 1:
            pending.append(issue(1, bufs[1]))
        out_h = [None, None, None]
        for c in range(nchunk):
            b = c % 3
            for h in pending.pop(0):
                h.wait()
            if c + 2 < nchunk:
                nb = (c + 2) % 3
                if out_h[nb] is not None:
                    out_h[nb].wait()
                    out_h[nb] = None
                pending.append(issue(c + 2, bufs[nb]))
            out_h[b] = pltpu.async_copy(
                bufs[b], out_hbm.at[pl.ds(base + c * 256, 256)], osems[b])
        for h in out_h:
            if h is not None:
                h.wait()

    return k(table_t, ids_chunk)


def _dense_tc(gathered_c, stats3, W_stat, b_stat, W_bot, b_comb,
              oct0, nocts, nl_real, prev_out=None):
    grid = (nocts, 4)
    g4 = gathered_c.reshape(nl_real, 4, BQ, D_MODEL)

    def body(*refs):
        if prev_out is None:
            g_ref, st_ref, ws_ref, bs_ref, wc_ref, bc_ref, out_ref = refs
        else:
            _, g_ref, st_ref, ws_ref, bs_ref, wc_ref, bc_ref, out_ref = refs
        for ll in range(8):
            x = st_ref[:, ll, :]
            s = jax.lax.dot_general(
                x, ws_ref[...],
                dimension_numbers=(((0,), (0,)), ((), ())),
                preferred_element_type=jnp.float32) + bs_ref[...]
            s = 0.5 * s * (1.0 + lax.erf(s * 0.7071067811865476))
            bot = jnp.dot(s, wc_ref[...], preferred_element_type=jnp.float32)
            out_ref[ll, 0] = g_ref[ll, 0] + bot + bc_ref[...]

    in_specs = [
        pl.BlockSpec((8, 1, BQ, D_MODEL), lambda li, bq: (li, bq, 0, 0)),
        pl.BlockSpec((10, 8, BQ), lambda li, bq: (0, oct0 + li, bq)),
        pl.BlockSpec((10, D_HALF), lambda li, bq: (0, 0)),
        pl.BlockSpec((1, D_HALF), lambda li, bq: (0, 0)),
        pl.BlockSpec((D_HALF, D_MODEL), lambda li, bq: (0, 0)),
        pl.BlockSpec((1, D_MODEL), lambda li, bq: (0, 0)),
    ]
    out_spec = pl.BlockSpec((8, 1, BQ, D_MODEL),
                            lambda li, bq: (oct0 + li, bq, 0, 0))
    operands = [g4, stats3, W_stat, b_stat.reshape(1, D_HALF),
                W_bot, b_comb.reshape(1, D_MODEL)]
    kwargs = {}
    if prev_out is not None:
        in_specs = [pl.BlockSpec(memory_space=pl.ANY)] + in_specs
        operands = [prev_out] + operands
        kwargs["input_output_aliases"] = {0: 0}

    return pl.pallas_call(
        body,
        grid=grid,
        in_specs=in_specs,
        out_specs=out_spec,
        out_shape=jax.ShapeDtypeStruct((L_DIM, 4, BQ, D_MODEL), jnp.float32),
        **kwargs,
    )(*operands)


def kernel(card_ids, card_stats, emb_table, W_stat, b_stat, W_comb, b_comb):
    B, L = card_ids.shape
    ids_flat = card_ids.T.reshape(N_TOKENS).astype(jnp.int32)
    stats3 = card_stats.transpose(2, 1, 0)
    W_bot = W_comb[D_HALF:, :]

    table_t = _table_transform_tc(emb_table.T, W_comb)

    nchunks = len(CHUNK_TOKS)
    gs = []
    off = 0
    for k in range(nchunks):
        gs.append(_gather_sc(table_t, ids_flat[off:off + CHUNK_TOKS[k]],
                             CHUNK_TOKS[k]))
        off += CHUNK_TOKS[k]

    out = None
    for k in range(nchunks):
        nl_real = CHUNK_TOKS[k] // B_DIM
        out = _dense_tc(gs[k], stats3, W_stat, b_stat, W_bot, b_comb,
                        CHUNK_L0[k] // 8, CHUNK_OCTS[k], nl_real,
                        prev_out=out)

    return out.reshape(L_DIM, B_DIM, D_MODEL).transpose(1, 0, 2)
